# Initial kernel scaffold; baseline (speedup 1.0000x reference)
import jax
import jax.numpy as jnp
from jax.experimental import pallas as pl


def kernel(input, dates, cmax, time_w, time_b, local_emb, space_emb):
    raise NotImplementedError("write your pallas kernel here")


# speedup vs baseline: 1.6091x; 1.6091x over previous
import jax
import jax.numpy as jnp
from jax.experimental import pallas as pl
from jax.experimental.pallas import tpu as pltpu

_B, _L, _DIN = 8, 2048, 16
_NT, _PD = 6, 6
_DM = 40
_T = _DIN * _L


def _body(inp_ref, dates_ref, cmax_ref, w2_ref, e3_ref, bias_ref, local_ref,
          space_ref, out_ref, var_ref, base_ref):
    d = pl.program_id(1)

    @pl.when(d == 0)
    def _():
        lin = (
            jnp.dot(dates_ref[0], w2_ref[...], preferred_element_type=jnp.float32)
            + jnp.dot(cmax_ref[0], e3_ref[...], preferred_element_type=jnp.float32)
            + bias_ref[...]
        )
        lane = jax.lax.broadcasted_iota(jnp.int32, (_L, _DM), 1)
        sinmask = (lane >= 1) & (lane <= 36) & ((lane - 1) % _PD != 0)
        base_ref[...] = local_ref[...] + jnp.where(sinmask, jnp.sin(lin), lin)

    lane16 = jax.lax.broadcasted_iota(jnp.int32, (_L, _DIN), 1)
    col0 = jnp.sum(jnp.where(lane16 == d, inp_ref[0], 0.0), axis=1, keepdims=True)
    srow = jax.lax.broadcasted_iota(jnp.int32, (_DIN, 1), 0)
    sval = jnp.sum(jnp.where(srow == d, space_ref[...], 0.0))
    lane40 = jax.lax.broadcasted_iota(jnp.int32, (_L, _DM), 1)
    out_ref[0] = base_ref[...] + jnp.where(lane40 == 0, col0 + sval, 0.0)
    var_ref[...] = jnp.zeros((1, 1, 1, _L), jnp.int32) + d


def kernel(input, dates, cmax, time_w, time_b, local_emb, space_emb):
    k = jnp.arange(_NT * _PD)
    w2p = jnp.zeros((_NT, _DM), jnp.float32)
    w2p = w2p.at[k // _PD, 1 + k].set(time_w.reshape(-1))
    bias = jnp.zeros((1, _DM), jnp.float32)
    bias = bias.at[0, 1 + k].set(time_b.reshape(-1))
    e3 = jnp.zeros((3, _DM), jnp.float32)
    e3 = e3.at[jnp.arange(3), 37 + jnp.arange(3)].set(1.0)

    out, var4 = pl.pallas_call(
        _body,
        grid=(_B, _DIN),
        in_specs=[
            pl.BlockSpec((1, _L, _DIN), lambda b, d: (b, 0, 0)),
            pl.BlockSpec((1, _L, _NT), lambda b, d: (b, 0, 0)),
            pl.BlockSpec((1, _L, 3), lambda b, d: (b, 0, 0)),
            pl.BlockSpec((_NT, _DM), lambda b, d: (0, 0)),
            pl.BlockSpec((3, _DM), lambda b, d: (0, 0)),
            pl.BlockSpec((1, _DM), lambda b, d: (0, 0)),
            pl.BlockSpec((_L, _DM), lambda b, d: (0, 0)),
            pl.BlockSpec((_DIN, 1), lambda b, d: (0, 0)),
        ],
        out_specs=[
            pl.BlockSpec((1, _L, _DM), lambda b, d: (b, d, 0)),
            pl.BlockSpec((1, 1, 1, _L), lambda b, d: (b, d, 0, 0)),
        ],
        out_shape=[
            jax.ShapeDtypeStruct((_B, _T, _DM), jnp.float32),
            jax.ShapeDtypeStruct((_B, _DIN, 1, _L), jnp.int32),
        ],
        scratch_shapes=[pltpu.VMEM((_L, _DM), jnp.float32)],
        compiler_params=pltpu.CompilerParams(
            dimension_semantics=("arbitrary", "arbitrary"),
        ),
    )(input, dates, cmax, w2p, e3, bias, local_emb[:_L], space_emb)
    return out, var4.reshape(_B, _T)


# speedup vs baseline: 4.7320x; 2.9409x over previous
import numpy as np
import jax
import jax.numpy as jnp
from jax.experimental import pallas as pl
from jax.experimental.pallas import tpu as pltpu

_B, _L, _DIN = 8, 2048, 16
_NT, _PD = 6, 6
_DM = 40
_T = _DIN * _L

_S40 = np.zeros((_DM, _NT), np.float32)
_S40[1 + np.arange(36), np.arange(36) // _PD] = 1.0
_S3 = np.zeros((_DM, 3), np.float32)
_S3[37 + np.arange(3), np.arange(3)] = 1.0


def _body(inp_ref, dates_ref, cmax_ref, w2_ref, s3_ref, bias_ref, local_ref,
          space_ref, out_ref, var_ref, base_ref):
    d = pl.program_id(1)

    @pl.when(d == 0)
    def _():
        lin = (
            jnp.dot(w2_ref[...], dates_ref[0], preferred_element_type=jnp.float32)
            + jnp.dot(s3_ref[...], cmax_ref[0], preferred_element_type=jnp.float32)
            + bias_ref[...]
        )
        row = jax.lax.broadcasted_iota(jnp.int32, (_DM, 1), 0)
        sinmask = (row >= 1) & (row <= 36) & ((row - 1) % _PD != 0)
        base_ref[...] = local_ref[...] + jnp.where(sinmask, jnp.sin(lin), lin)
        var_ref[0] = jax.lax.broadcasted_iota(jnp.int32, (_DIN, _L), 0)

    sub16 = jax.lax.broadcasted_iota(jnp.int32, (_DIN, _L), 0)
    col0 = jnp.sum(jnp.where(sub16 == d, inp_ref[0], 0.0), axis=0, keepdims=True)
    srow = jax.lax.broadcasted_iota(jnp.int32, (_DIN, 1), 0)
    sval = jnp.sum(jnp.where(srow == d, space_ref[...], 0.0))
    row40 = jax.lax.broadcasted_iota(jnp.int32, (_DM, 1), 0)
    out_ref[0] = base_ref[...] + jnp.where(row40 == 0, col0 + sval, 0.0)


def kernel(input, dates, cmax, time_w, time_b, local_emb, space_emb):
    inp_t = jnp.transpose(input, (0, 2, 1))
    dates_t = jnp.transpose(dates, (0, 2, 1))
    cmax_t = jnp.transpose(cmax, (0, 2, 1))
    local_t = jnp.transpose(local_emb[:_L])

    w_flat = jnp.pad(time_w.reshape(-1), (1, 3))
    b_flat = jnp.pad(time_b.reshape(-1), (1, 3))
    w2t = jnp.asarray(_S40) * w_flat[:, None]
    bias = b_flat[:, None]

    out_cm, var3 = pl.pallas_call(
        _body,
        grid=(_B, _DIN),
        in_specs=[
            pl.BlockSpec((1, _DIN, _L), lambda b, d: (b, 0, 0)),
            pl.BlockSpec((1, _NT, _L), lambda b, d: (b, 0, 0)),
            pl.BlockSpec((1, 3, _L), lambda b, d: (b, 0, 0)),
            pl.BlockSpec((_DM, _NT), lambda b, d: (0, 0)),
            pl.BlockSpec((_DM, 3), lambda b, d: (0, 0)),
            pl.BlockSpec((_DM, 1), lambda b, d: (0, 0)),
            pl.BlockSpec((_DM, _L), lambda b, d: (0, 0)),
            pl.BlockSpec((_DIN, 1), lambda b, d: (0, 0)),
        ],
        out_specs=[
            pl.BlockSpec((1, _DM, _L), lambda b, d: (b, 0, d)),
            pl.BlockSpec((1, _DIN, _L), lambda b, d: (b, 0, 0)),
        ],
        out_shape=[
            jax.ShapeDtypeStruct((_B, _DM, _T), jnp.float32),
            jax.ShapeDtypeStruct((_B, _DIN, _L), jnp.int32),
        ],
        scratch_shapes=[pltpu.VMEM((_DM, _L), jnp.float32)],
        compiler_params=pltpu.CompilerParams(
            dimension_semantics=("arbitrary", "arbitrary"),
        ),
    )(inp_t, dates_t, cmax_t, w2t, jnp.asarray(_S3), bias, local_t, space_emb)
    out = jnp.transpose(out_cm, (0, 2, 1))
    return out, var3.reshape(_B, _T)


# speedup vs baseline: 13.9153x; 2.9407x over previous
import numpy as np
import jax
import jax.numpy as jnp
from jax.experimental import pallas as pl
from jax.experimental.pallas import tpu as pltpu

_B, _L, _DIN = 8, 2048, 16
_NT, _PD = 6, 6
_DM = 40
_T = _DIN * _L

_K = np.arange(36)
_U = np.zeros((_DM, _NT), np.float32)
_U[1 + _K, _K // _PD] = 1.0
_V = np.zeros((_NT, _DM), np.float32)
_V[_K % _PD, 1 + _K] = 1.0
_S3T = np.zeros((_DM, 3), np.float32)
_S3T[37 + np.arange(3), np.arange(3)] = 1.0


def _body(inp_ref, dates_ref, cmax_ref, tw_ref, tb_ref, u_ref, v_ref, s3t_ref,
          local_ref, space_ref, out_ref, var_ref, base_ref):
    u = u_ref[...]
    v = v_ref[...]
    dw = jnp.dot(jnp.dot(u, tw_ref[...], preferred_element_type=jnp.float32),
                 v, preferred_element_type=jnp.float32)
    db = jnp.dot(jnp.dot(u, tb_ref[...], preferred_element_type=jnp.float32),
                 v, preferred_element_type=jnp.float32)
    r0 = jax.lax.broadcasted_iota(jnp.int32, (_DM, _DM), 0)
    r1 = jax.lax.broadcasted_iota(jnp.int32, (_DM, _DM), 1)
    eye = (r0 == r1).astype(jnp.float32)
    w40 = jnp.sum(dw * eye, axis=1, keepdims=True)
    b40 = jnp.sum(db * eye, axis=1, keepdims=True)
    dates_spread = jnp.dot(u, dates_ref[0],
                           preferred_element_type=jnp.float32)
    lin = (dates_spread * w40 + b40
           + jnp.dot(s3t_ref[...], cmax_ref[0],
                     preferred_element_type=jnp.float32))
    row = jax.lax.broadcasted_iota(jnp.int32, (_DM, 1), 0)
    sinmask = (row >= 1) & (row <= 36) & ((row - 1) % _PD != 0)
    base_ref[...] = local_ref[...] + jnp.where(sinmask, jnp.sin(lin), lin)
    var_ref[0] = jax.lax.broadcasted_iota(jnp.int32, (_DIN, _L), 0)

    base0 = base_ref[0:1, :]
    for d in range(_DIN):
        sl = pl.ds(d * _L, _L)
        out_ref[0, :, sl] = base_ref[...]
        out_ref[0, 0:1, sl] = (base0 + inp_ref[0, d:d + 1, :]
                               + space_ref[d:d + 1, :])


def kernel(input, dates, cmax, time_w, time_b, local_emb, space_emb):
    inp_t = jnp.transpose(input, (0, 2, 1))
    dates_t = jnp.transpose(dates, (0, 2, 1))
    cmax_t = jnp.transpose(cmax, (0, 2, 1))
    local_t = jnp.transpose(local_emb)

    out_cm, var3 = pl.pallas_call(
        _body,
        grid=(_B,),
        in_specs=[
            pl.BlockSpec((1, _DIN, _L), lambda b: (b, 0, 0)),
            pl.BlockSpec((1, _NT, _L), lambda b: (b, 0, 0)),
            pl.BlockSpec((1, 3, _L), lambda b: (b, 0, 0)),
            pl.BlockSpec((_NT, _NT), lambda b: (0, 0)),
            pl.BlockSpec((_NT, _NT), lambda b: (0, 0)),
            pl.BlockSpec((_DM, _NT), lambda b: (0, 0)),
            pl.BlockSpec((_NT, _DM), lambda b: (0, 0)),
            pl.BlockSpec((_DM, 3), lambda b: (0, 0)),
            pl.BlockSpec((_DM, _L), lambda b: (0, 0)),
            pl.BlockSpec((_DIN, 1), lambda b: (0, 0)),
        ],
        out_specs=[
            pl.BlockSpec((1, _DM, _T), lambda b: (b, 0, 0)),
            pl.BlockSpec((1, _DIN, _L), lambda b: (b, 0, 0)),
        ],
        out_shape=[
            jax.ShapeDtypeStruct((_B, _DM, _T), jnp.float32),
            jax.ShapeDtypeStruct((_B, _DIN, _L), jnp.int32),
        ],
        scratch_shapes=[pltpu.VMEM((_DM, _L), jnp.float32)],
        compiler_params=pltpu.CompilerParams(
            dimension_semantics=("arbitrary",),
            vmem_limit_bytes=50 * 1024 * 1024,
        ),
    )(inp_t, dates_t, cmax_t, time_w, time_b, jnp.asarray(_U), jnp.asarray(_V),
      jnp.asarray(_S3T), local_t, space_emb)
    out = jnp.transpose(out_cm, (0, 2, 1))
    return out, var3.reshape(_B, _T)


# speedup vs baseline: 16.1380x; 1.1597x over previous
import jax
import jax.numpy as jnp
from jax.experimental import pallas as pl
from jax.experimental.pallas import tpu as pltpu

_B, _L, _DIN = 8, 2048, 16
_NT, _PD = 6, 6
_DM = 40
_T = _DIN * _L


def _f32(x):
    return x.astype(jnp.float32)


def _body(inp_ref, dates_ref, cmax_ref, tw_ref, tb_ref, local_ref, space_ref,
          out_ref, var_ref, base_ref):
    b = pl.program_id(0)

    @pl.when(b == 0)
    def _():
        tvar = jax.lax.broadcasted_iota(jnp.int32, (_B, _T), 1)
        var_ref[...] = jax.lax.shift_right_logical(tvar, 11)

    c40_6 = jax.lax.broadcasted_iota(jnp.int32, (_DM, _NT), 0)
    i40_6 = jax.lax.broadcasted_iota(jnp.int32, (_DM, _NT), 1)
    trow = (c40_6 >= 1) & (c40_6 <= 36)
    u = _f32(trow & (i40_6 == (c40_6 - 1) // _PD))
    j6_40 = jax.lax.broadcasted_iota(jnp.int32, (_NT, _DM), 0)
    c6_40 = jax.lax.broadcasted_iota(jnp.int32, (_NT, _DM), 1)
    v = _f32((c6_40 >= 1) & (c6_40 <= 36) & (j6_40 == (c6_40 - 1) % _PD))
    c40_3 = jax.lax.broadcasted_iota(jnp.int32, (_DM, 3), 0)
    h40_3 = jax.lax.broadcasted_iota(jnp.int32, (_DM, 3), 1)
    s3t = _f32(c40_3 == 37 + h40_3)

    bsub6 = jax.lax.broadcasted_iota(jnp.int32, (_NT, _B, _L), 1)
    dates_b = jnp.sum(jnp.where(bsub6 == b, dates_ref[...], 0.0), axis=1)
    bsub3 = jax.lax.broadcasted_iota(jnp.int32, (3, _B, _L), 1)
    cmax_b = jnp.sum(jnp.where(bsub3 == b, cmax_ref[...], 0.0), axis=1)

    dw = jnp.dot(jnp.dot(u, tw_ref[...], preferred_element_type=jnp.float32),
                 v, preferred_element_type=jnp.float32)
    db = jnp.dot(jnp.dot(u, tb_ref[...], preferred_element_type=jnp.float32),
                 v, preferred_element_type=jnp.float32)
    r0 = jax.lax.broadcasted_iota(jnp.int32, (_DM, _DM), 0)
    r1 = jax.lax.broadcasted_iota(jnp.int32, (_DM, _DM), 1)
    eye = _f32(r0 == r1)
    w40 = jnp.sum(dw * eye, axis=1, keepdims=True)
    b40 = jnp.sum(db * eye, axis=1, keepdims=True)

    dates_spread = jnp.dot(u, dates_b, preferred_element_type=jnp.float32)
    lin = (dates_spread * w40 + b40
           + jnp.dot(s3t, cmax_b, preferred_element_type=jnp.float32))
    row = jax.lax.broadcasted_iota(jnp.int32, (_DM, 1), 0)
    sinmask = (row >= 1) & (row <= 36) & ((row - 1) % _PD != 0)
    base_ref[...] = local_ref[...] + jnp.where(sinmask, jnp.sin(lin), lin)

    base0 = base_ref[0:1, :]
    for d in range(_DIN):
        sl = pl.ds(d * _L, _L)
        out_ref[0, :, sl] = base_ref[...]
        out_ref[0, 0:1, sl] = (base0 + inp_ref[0, d:d + 1, :]
                               + space_ref[0:1, d:d + 1])


def kernel(input, dates, cmax, time_w, time_b, local_emb, space_emb):
    inp_t = jnp.transpose(input, (0, 2, 1))
    dates_t = jnp.transpose(dates, (2, 0, 1))
    cmax_t = jnp.transpose(cmax, (2, 0, 1))
    local_t = jnp.transpose(local_emb)
    space_t = jnp.transpose(space_emb)

    out_cm, var = pl.pallas_call(
        _body,
        grid=(_B,),
        in_specs=[
            pl.BlockSpec((1, _DIN, _L), lambda b: (b, 0, 0)),
            pl.BlockSpec((_NT, _B, _L), lambda b: (0, 0, 0)),
            pl.BlockSpec((3, _B, _L), lambda b: (0, 0, 0)),
            pl.BlockSpec((_NT, _NT), lambda b: (0, 0)),
            pl.BlockSpec((_NT, _NT), lambda b: (0, 0)),
            pl.BlockSpec((_DM, _L), lambda b: (0, 0)),
            pl.BlockSpec((1, _DIN), lambda b: (0, 0)),
        ],
        out_specs=[
            pl.BlockSpec((1, _DM, _T), lambda b: (b, 0, 0)),
            pl.BlockSpec((_B, _T), lambda b: (0, 0)),
        ],
        out_shape=[
            jax.ShapeDtypeStruct((_B, _DM, _T), jnp.float32),
            jax.ShapeDtypeStruct((_B, _T), jnp.int32),
        ],
        scratch_shapes=[pltpu.VMEM((_DM, _L), jnp.float32)],
        compiler_params=pltpu.CompilerParams(
            dimension_semantics=("arbitrary",),
            vmem_limit_bytes=50 * 1024 * 1024,
        ),
    )(inp_t, dates_t, cmax_t, time_w, time_b, local_t, space_t)
    out = jnp.transpose(out_cm, (0, 2, 1))
    return out, var
